# initial kernel scaffold (unmeasured)
import jax
import jax.numpy as jnp
from jax import lax
from jax.experimental import pallas as pl
from jax.experimental.pallas import tpu as pltpu

M_BLK = 2048
Q = 1024
D = 2048


def kernel(partial, gamma):
    gamma2d = gamma.reshape(1, D)

    def body(partial_ref, gamma_ref, out_ref, stage_ref, xsend_ref, xrecv_ref,
             ysend_ref, yrecv_ref, local_sem, sems):
        my_x = lax.axis_index("x")
        my_y = lax.axis_index("y")
        xn = (1 - my_x, my_y)
        yn = (my_x, 1 - my_y)

        barrier = pltpu.get_barrier_semaphore()
        for nbr in (xn, yn):
            pl.semaphore_signal(barrier, inc=1, device_id=nbr,
                                device_id_type=pl.DeviceIdType.MESH)
        pl.semaphore_wait(barrier, 2)

        base_send = (1 - my_x) * M_BLK + my_y * Q
        base_q = my_x * M_BLK + my_y * Q

        cp = pltpu.make_async_copy(
            partial_ref.at[0, pl.ds(base_send, Q), :], stage_ref, local_sem)
        cp.start()
        cp.wait()
        xsend_ref[...] = stage_ref[...].astype(jnp.bfloat16)

        rdma_x = pltpu.make_async_remote_copy(
            src_ref=xsend_ref, dst_ref=xrecv_ref,
            send_sem=sems.at[0], recv_sem=sems.at[1],
            device_id=xn, device_id_type=pl.DeviceIdType.MESH)
        rdma_x.start()

        cp2 = pltpu.make_async_copy(
            partial_ref.at[0, pl.ds(base_q, Q), :], stage_ref, local_sem)
        cp2.start()
        cp2.wait()

        rdma_x.wait()

        s = stage_ref[...] + xrecv_ref[...].astype(jnp.float32)
        rms = jnp.sqrt(jnp.mean(s * s, axis=-1, keepdims=True) + 1e-6)
        o = s / rms * gamma_ref[...]
        out_ref[pl.ds(my_y * Q, Q), :] = o
        ysend_ref[...] = o.astype(jnp.bfloat16)

        rdma_y = pltpu.make_async_remote_copy(
            src_ref=ysend_ref, dst_ref=yrecv_ref,
            send_sem=sems.at[2], recv_sem=sems.at[3],
            device_id=yn, device_id_type=pl.DeviceIdType.MESH)
        rdma_y.start()
        rdma_y.wait()

        out_ref[pl.ds((1 - my_y) * Q, Q), :] = yrecv_ref[...].astype(jnp.float32)

    return pl.pallas_call(
        body,
        out_shape=jax.ShapeDtypeStruct((M_BLK, D), jnp.float32),
        in_specs=[
            pl.BlockSpec(memory_space=pltpu.ANY),
            pl.BlockSpec(memory_space=pltpu.VMEM),
        ],
        out_specs=pl.BlockSpec(memory_space=pltpu.VMEM),
        scratch_shapes=[
            pltpu.VMEM((Q, D), jnp.float32),
            pltpu.VMEM((Q, D), jnp.bfloat16),
            pltpu.VMEM((Q, D), jnp.bfloat16),
            pltpu.VMEM((Q, D), jnp.bfloat16),
            pltpu.VMEM((Q, D), jnp.bfloat16),
            pltpu.SemaphoreType.DMA,
            pltpu.SemaphoreType.DMA((4,)),
        ],
        compiler_params=pltpu.CompilerParams(collective_id=0),
    )(partial, gamma2d)


# baseline (device time: 122314 ns/iter reference)
import jax
import jax.numpy as jnp
from jax import lax
from jax.experimental import pallas as pl
from jax.experimental.pallas import tpu as pltpu

M_BLK = 2048
Q = 1024
D = 2048


def kernel(partial, gamma):
    gamma2d = gamma.reshape(1, D)

    def body(partial_ref, gamma_ref, out_ref, stage_ref, xsend_ref, xrecv_ref,
             ysend_ref, yrecv_ref, local_sem, sems):
        my_x = lax.axis_index("x")
        my_y = lax.axis_index("y")
        xn = (1 - my_x, my_y)
        yn = (my_x, 1 - my_y)

        barrier = pltpu.get_barrier_semaphore()
        for nbr in (xn, yn):
            pl.semaphore_signal(barrier, inc=1, device_id=nbr,
                                device_id_type=pl.DeviceIdType.MESH)
        pl.semaphore_wait(barrier, 2)

        base_send = (1 - my_x) * M_BLK + my_y * Q
        base_q = my_x * M_BLK + my_y * Q

        cp = pltpu.make_async_copy(
            partial_ref.at[0, pl.ds(base_send, Q), :], stage_ref, local_sem)
        cp.start()
        cp.wait()
        xsend_ref[...] = stage_ref[...].astype(jnp.bfloat16)

        rdma_x = pltpu.make_async_remote_copy(
            src_ref=xsend_ref, dst_ref=xrecv_ref,
            send_sem=sems.at[0], recv_sem=sems.at[1],
            device_id=xn, device_id_type=pl.DeviceIdType.MESH)
        rdma_x.start()

        cp2 = pltpu.make_async_copy(
            partial_ref.at[0, pl.ds(base_q, Q), :], stage_ref, local_sem)
        cp2.start()
        cp2.wait()

        rdma_x.wait()

        s = stage_ref[...] + xrecv_ref[...].astype(jnp.float32)
        rms = jnp.sqrt(jnp.mean(s * s, axis=-1, keepdims=True) + 1e-6)
        o = s / rms * gamma_ref[...]
        out_ref[pl.ds(my_y * Q, Q), :] = o
        ysend_ref[...] = o.astype(jnp.bfloat16)

        rdma_y = pltpu.make_async_remote_copy(
            src_ref=ysend_ref, dst_ref=yrecv_ref,
            send_sem=sems.at[2], recv_sem=sems.at[3],
            device_id=yn, device_id_type=pl.DeviceIdType.MESH)
        rdma_y.start()
        rdma_y.wait()

        out_ref[pl.ds((1 - my_y) * Q, Q), :] = yrecv_ref[...].astype(jnp.float32)

    return pl.pallas_call(
        body,
        out_shape=jax.ShapeDtypeStruct((M_BLK, D), jnp.float32),
        in_specs=[
            pl.BlockSpec(memory_space=pltpu.MemorySpace.HBM),
            pl.BlockSpec(memory_space=pltpu.VMEM),
        ],
        out_specs=pl.BlockSpec(memory_space=pltpu.VMEM),
        scratch_shapes=[
            pltpu.VMEM((Q, D), jnp.float32),
            pltpu.VMEM((Q, D), jnp.bfloat16),
            pltpu.VMEM((Q, D), jnp.bfloat16),
            pltpu.VMEM((Q, D), jnp.bfloat16),
            pltpu.VMEM((Q, D), jnp.bfloat16),
            pltpu.SemaphoreType.DMA,
            pltpu.SemaphoreType.DMA((4,)),
        ],
        compiler_params=pltpu.CompilerParams(
            collective_id=0, vmem_limit_bytes=128 * 1024 * 1024
        ),
    )(partial, gamma2d)


# device time: 80126 ns/iter; 1.5265x vs baseline; 1.5265x over previous
import jax
import jax.numpy as jnp
from jax import lax
from jax.experimental import pallas as pl
from jax.experimental.pallas import tpu as pltpu

M_BLK = 2048
Q = 1024
D = 2048
C = 8
CR = Q // C


def kernel(partial, gamma):
    gamma2d = gamma.reshape(1, D)

    def body(partial_ref, gamma_ref, out_ref, stage_send, stage_local,
             xsend_ref, xrecv_ref, ysend_ref, yrecv_ref, local_sems,
             xs_sems, xr_sems, ys_sems, yr_sems):
        my_x = lax.axis_index("x")
        my_y = lax.axis_index("y")
        xn = (1 - my_x, my_y)
        yn = (my_x, 1 - my_y)

        barrier = pltpu.get_barrier_semaphore()
        for nbr in (xn, yn):
            pl.semaphore_signal(barrier, inc=1, device_id=nbr,
                                device_id_type=pl.DeviceIdType.MESH)
        pl.semaphore_wait(barrier, 2)

        base_send = (1 - my_x) * M_BLK + my_y * Q
        base_q = my_x * M_BLK + my_y * Q

        cp_send = pltpu.make_async_copy(
            partial_ref.at[0, pl.ds(base_send, Q), :], stage_send,
            local_sems.at[0])
        cp_send.start()
        cp_loc = pltpu.make_async_copy(
            partial_ref.at[0, pl.ds(base_q, Q), :], stage_local,
            local_sems.at[1])
        cp_loc.start()

        cp_send.wait()
        rdx = []
        for i in range(C):
            sl = pl.ds(i * CR, CR)
            xsend_ref[sl, :] = stage_send[sl, :].astype(jnp.bfloat16)
            r = pltpu.make_async_remote_copy(
                src_ref=xsend_ref.at[sl, :], dst_ref=xrecv_ref.at[sl, :],
                send_sem=xs_sems.at[i], recv_sem=xr_sems.at[i],
                device_id=xn, device_id_type=pl.DeviceIdType.MESH)
            r.start()
            rdx.append(r)

        cp_loc.wait()
        rdy = []
        for i in range(C):
            sl = pl.ds(i * CR, CR)
            rdx[i].wait_recv()
            s = stage_local[sl, :] + xrecv_ref[sl, :].astype(jnp.float32)
            rms = jnp.sqrt(jnp.mean(s * s, axis=-1, keepdims=True) + 1e-6)
            o = s / rms * gamma_ref[...]
            out_ref[pl.ds(my_y * Q + i * CR, CR), :] = o
            ysend_ref[sl, :] = o.astype(jnp.bfloat16)
            ry = pltpu.make_async_remote_copy(
                src_ref=ysend_ref.at[sl, :], dst_ref=yrecv_ref.at[sl, :],
                send_sem=ys_sems.at[i], recv_sem=yr_sems.at[i],
                device_id=yn, device_id_type=pl.DeviceIdType.MESH)
            ry.start()
            rdy.append(ry)

        for i in range(C):
            sl = pl.ds(i * CR, CR)
            rdy[i].wait_recv()
            out_ref[pl.ds((1 - my_y) * Q + i * CR, CR), :] = (
                yrecv_ref[sl, :].astype(jnp.float32))

        for i in range(C):
            rdx[i].wait_send()
            rdy[i].wait_send()

    return pl.pallas_call(
        body,
        out_shape=jax.ShapeDtypeStruct((M_BLK, D), jnp.float32),
        in_specs=[
            pl.BlockSpec(memory_space=pltpu.MemorySpace.HBM),
            pl.BlockSpec(memory_space=pltpu.VMEM),
        ],
        out_specs=pl.BlockSpec(memory_space=pltpu.VMEM),
        scratch_shapes=[
            pltpu.VMEM((Q, D), jnp.float32),
            pltpu.VMEM((Q, D), jnp.float32),
            pltpu.VMEM((Q, D), jnp.bfloat16),
            pltpu.VMEM((Q, D), jnp.bfloat16),
            pltpu.VMEM((Q, D), jnp.bfloat16),
            pltpu.VMEM((Q, D), jnp.bfloat16),
            pltpu.SemaphoreType.DMA((2,)),
            pltpu.SemaphoreType.DMA((C,)),
            pltpu.SemaphoreType.DMA((C,)),
            pltpu.SemaphoreType.DMA((C,)),
            pltpu.SemaphoreType.DMA((C,)),
        ],
        compiler_params=pltpu.CompilerParams(
            collective_id=0, vmem_limit_bytes=128 * 1024 * 1024
        ),
    )(partial, gamma2d)
